# baseline (speedup 1.0000x reference)
import functools

import jax
import jax.numpy as jnp
from jax import lax
from jax.experimental import pallas as pl
from jax.experimental.pallas import tpu as pltpu
from jax.experimental.pallas import tpu_sc as plsc

NC = 2
NS = 16
L = 16
NW = NC * NS
LANES = 128
DEPTH = 4
OUTW = 1024


def _make_bpr(B, D):
    bpw = B // NW
    n_chunks = bpw // L
    mesh = plsc.VectorSubcoreMesh(core_axis_name="c", subcore_axis_name="s")

    @functools.partial(
        pl.kernel,
        out_type=jax.ShapeDtypeStruct((NW * OUTW,), jnp.float32),
        mesh=mesh,
        compiler_params=pltpu.CompilerParams(needs_layout_passes=False),
        scratch_types=[
            pltpu.VMEM((2 * bpw,), jnp.int32),
            pltpu.VMEM((2 * bpw,), jnp.int32),
            pltpu.VMEM((2 * bpw,), jnp.float32),
            pltpu.VMEM((DEPTH, D, LANES), jnp.float32),
            pltpu.VMEM((DEPTH, D, LANES), jnp.float32),
            pltpu.VMEM((OUTW,), jnp.float32),
        ] + [pltpu.SemaphoreType.DMA] * (2 * DEPTH),
    )
    def bpr(users_hbm, items_hbm, scores_hbm, utT_hbm, itT_hbm, out_hbm,
            uidx, iidx, sc_v, ublk, iblk, acc_v, *sems):
        cid = lax.axis_index("c")
        sid = lax.axis_index("s")
        wid = sid * NC + cid
        blk = (wid // 2) * (2 * bpw)
        loc = (wid % 2) * bpw
        pltpu.sync_copy(users_hbm.at[pl.ds(blk, 2 * bpw)], uidx)
        pltpu.sync_copy(items_hbm.at[pl.ds(blk, 2 * bpw)], iidx)
        pltpu.sync_copy(scores_hbm.at[pl.ds(blk, 2 * bpw)], sc_v)

        usem = sems[:DEPTH]
        isem = sems[DEPTH:]
        lane = lax.iota(jnp.int32, L)

        def fire(r, slot, uval, ival):
            uq = pl.multiple_of(
                lax.shift_right_logical(uval, 7) * LANES, LANES)
            iq = pl.multiple_of(
                lax.shift_right_logical(ival, 7) * LANES, LANES)
            pltpu.async_copy(
                utT_hbm.at[:, pl.ds(uq, LANES)], ublk.at[slot], usem[slot])
            pltpu.async_copy(
                itT_hbm.at[:, pl.ds(iq, LANES)], iblk.at[slot], isem[slot])

        def drain(slot):
            pltpu.make_async_copy(
                utT_hbm.at[:, pl.ds(0, LANES)], ublk.at[slot],
                usem[slot]).wait()
            pltpu.make_async_copy(
                itT_hbm.at[:, pl.ds(0, LANES)], iblk.at[slot],
                isem[slot]).wait()

        def compute(slot, uval, ival, sval, acc):
            ul = jnp.full((L,), uval & (LANES - 1), jnp.int32)
            il = jnp.full((L,), ival & (LANES - 1), jnp.int32)
            prod = jnp.zeros((L,), jnp.float32)
            for h in range(D // L):
                rows = h * L + lane
                gu = plsc.load_gather(ublk.at[slot], [rows, ul])
                gi = plsc.load_gather(iblk.at[slot], [rows, il])
                prod = prod + gu * gi
            diff = jnp.sum(prod) - sval
            return acc + diff * diff

        u0 = uidx[pl.ds(loc, L)]
        i0 = iidx[pl.ds(loc, L)]
        for k in range(DEPTH):
            fire(k, k % DEPTH, u0[k], i0[k])

        def step(c, acc):
            off = loc + c * L
            offn = jnp.minimum(off + L, 2 * bpw - L)
            u16 = uidx[pl.ds(off, L)]
            i16 = iidx[pl.ds(off, L)]
            u16n = uidx[pl.ds(offn, L)]
            i16n = iidx[pl.ds(offn, L)]
            scv = sc_v[pl.ds(off, L)]
            for k in range(L):
                r = c * L + k
                ka = k + DEPTH
                if ka < L:
                    ua, ia = u16[ka], i16[ka]
                else:
                    ua, ia = u16n[ka - L], i16n[ka - L]
                slot = k % DEPTH
                drain(slot)
                acc = compute(slot, u16[k], i16[k], scv[k], acc)

                @pl.when(r + DEPTH < bpw)
                def _():
                    fire(r + DEPTH, slot, ua, ia)
            return acc

        acc = lax.fori_loop(0, n_chunks, step, jnp.float32(0.0))

        zero = jnp.zeros((L,), jnp.float32)
        acc_v[pl.ds(0, L)] = jnp.where(lane == 0, acc, jnp.float32(0.0))
        for g in range(1, OUTW // L):
            acc_v[pl.ds(g * L, L)] = zero
        pltpu.sync_copy(acc_v, out_hbm.at[pl.ds(wid * OUTW, OUTW)])

    return bpr


def kernel(users, items, scores, user_table, item_table):
    B = users.shape[0]
    D = user_table.shape[1]
    bpr = _make_bpr(B, D)
    partials = bpr(users.astype(jnp.int32), items.astype(jnp.int32),
                   scores, user_table.T, item_table.T)
    return jnp.sum(partials) / B


# speedup vs baseline: 1.0063x; 1.0063x over previous
import functools

import jax
import jax.numpy as jnp
from jax import lax
from jax.experimental import pallas as pl
from jax.experimental.pallas import tpu as pltpu
from jax.experimental.pallas import tpu_sc as plsc

NC = 2
NS = 16
L = 16
NW = NC * NS
LANES = 128
DEPTH = 8
OUTW = 1024


def _make_bpr(B, D):
    bpw = B // NW
    n_chunks = bpw // L
    mesh = plsc.VectorSubcoreMesh(core_axis_name="c", subcore_axis_name="s")

    @functools.partial(
        pl.kernel,
        out_type=jax.ShapeDtypeStruct((NW * OUTW,), jnp.float32),
        mesh=mesh,
        compiler_params=pltpu.CompilerParams(needs_layout_passes=False),
        scratch_types=[
            pltpu.VMEM((2 * bpw,), jnp.int32),
            pltpu.VMEM((2 * bpw,), jnp.int32),
            pltpu.VMEM((2 * bpw,), jnp.float32),
            pltpu.VMEM((DEPTH, D, LANES), jnp.float32),
            pltpu.VMEM((DEPTH, D, LANES), jnp.float32),
            pltpu.VMEM((OUTW,), jnp.float32),
        ] + [pltpu.SemaphoreType.DMA] * (2 * DEPTH),
    )
    def bpr(users_hbm, items_hbm, scores_hbm, utT_hbm, itT_hbm, out_hbm,
            uidx, iidx, sc_v, ublk, iblk, acc_v, *sems):
        cid = lax.axis_index("c")
        sid = lax.axis_index("s")
        wid = sid * NC + cid
        blk = (wid // 2) * (2 * bpw)
        loc = (wid % 2) * bpw
        pltpu.sync_copy(users_hbm.at[pl.ds(blk, 2 * bpw)], uidx)
        pltpu.sync_copy(items_hbm.at[pl.ds(blk, 2 * bpw)], iidx)
        pltpu.sync_copy(scores_hbm.at[pl.ds(blk, 2 * bpw)], sc_v)

        usem = sems[:DEPTH]
        isem = sems[DEPTH:]
        lane = lax.iota(jnp.int32, L)

        def fire(r, slot, uval, ival):
            uq = pl.multiple_of(
                lax.shift_right_logical(uval, 7) * LANES, LANES)
            iq = pl.multiple_of(
                lax.shift_right_logical(ival, 7) * LANES, LANES)
            pltpu.async_copy(
                utT_hbm.at[:, pl.ds(uq, LANES)], ublk.at[slot], usem[slot])
            pltpu.async_copy(
                itT_hbm.at[:, pl.ds(iq, LANES)], iblk.at[slot], isem[slot])

        def drain(slot):
            pltpu.make_async_copy(
                utT_hbm.at[:, pl.ds(0, LANES)], ublk.at[slot],
                usem[slot]).wait()
            pltpu.make_async_copy(
                itT_hbm.at[:, pl.ds(0, LANES)], iblk.at[slot],
                isem[slot]).wait()

        def compute(slot, uval, ival, sval, acc):
            ul = jnp.full((L,), uval & (LANES - 1), jnp.int32)
            il = jnp.full((L,), ival & (LANES - 1), jnp.int32)
            prod = jnp.zeros((L,), jnp.float32)
            for h in range(D // L):
                rows = h * L + lane
                gu = plsc.load_gather(ublk.at[slot], [rows, ul])
                gi = plsc.load_gather(iblk.at[slot], [rows, il])
                prod = prod + gu * gi
            diff = jnp.sum(prod) - sval
            return acc + diff * diff

        u0 = uidx[pl.ds(loc, L)]
        i0 = iidx[pl.ds(loc, L)]
        for k in range(DEPTH):
            fire(k, k % DEPTH, u0[k], i0[k])

        def step(c, acc):
            off = loc + c * L
            offn = jnp.minimum(off + L, 2 * bpw - L)
            u16 = uidx[pl.ds(off, L)]
            i16 = iidx[pl.ds(off, L)]
            u16n = uidx[pl.ds(offn, L)]
            i16n = iidx[pl.ds(offn, L)]
            scv = sc_v[pl.ds(off, L)]
            for k in range(L):
                r = c * L + k
                ka = k + DEPTH
                if ka < L:
                    ua, ia = u16[ka], i16[ka]
                else:
                    ua, ia = u16n[ka - L], i16n[ka - L]
                slot = k % DEPTH
                drain(slot)
                acc = compute(slot, u16[k], i16[k], scv[k], acc)

                @pl.when(r + DEPTH < bpw)
                def _():
                    fire(r + DEPTH, slot, ua, ia)
            return acc

        acc = lax.fori_loop(0, n_chunks, step, jnp.float32(0.0))

        zero = jnp.zeros((L,), jnp.float32)
        acc_v[pl.ds(0, L)] = jnp.where(lane == 0, acc, jnp.float32(0.0))
        for g in range(1, OUTW // L):
            acc_v[pl.ds(g * L, L)] = zero
        pltpu.sync_copy(acc_v, out_hbm.at[pl.ds(wid * OUTW, OUTW)])

    return bpr


def kernel(users, items, scores, user_table, item_table):
    B = users.shape[0]
    D = user_table.shape[1]
    bpr = _make_bpr(B, D)
    partials = bpr(users.astype(jnp.int32), items.astype(jnp.int32),
                   scores, user_table.T, item_table.T)
    return jnp.sum(partials) / B
